# Initial kernel scaffold; baseline (speedup 1.0000x reference)
import jax
import jax.numpy as jnp
from jax.experimental import pallas as pl


def kernel(x, edge_index_parent, edge_index_child, edge_index_relate, W1p, b1p, W1c, b1c, W1r, b1r, W2p, b2p, W2c, b2c, W2r, b2r):
    raise NotImplementedError("write your pallas kernel here")


# speedup vs baseline: 16.5660x; 16.5660x over previous
import functools

import jax
import jax.numpy as jnp
from jax import lax
from jax.experimental import pallas as pl
from jax.experimental.pallas import tpu as pltpu
from jax.experimental.pallas import tpu_sc as plsc

N = 10000
D = 128
E = 320000

NC = 2
NS = 16
NW = NC * NS

NPAD = 10240
RPT = NPAD // NS
C = 80
EPW = E // NW
NCH = EPW // C
NROW = E // C

BN = 640
GRID = NPAD // BN

_mesh = plsc.VectorSubcoreMesh(core_axis_name="c", subcore_axis_name="s")


def _sc_degree_body(d0, d1, d2, zn, out, idx_v, ones_v, a0, a1, a2):
    cid = lax.axis_index("c")
    sid = lax.axis_index("s")
    wid = sid * NC + cid
    accs = (a0, a1, a2)
    for j in range(C // 16):
        ones_v[pl.ds(16 * j, 16)] = jnp.ones((16,), jnp.float32)
    for acc in accs:
        pltpu.sync_copy(zn.at[pl.ds(sid * RPT, RPT)],
                        acc.at[pl.ds(sid * RPT, RPT)])
    plsc.subcore_barrier()
    for r, dref in enumerate((d0, d1, d2)):
        acc = accs[r]
        pltpu.sync_copy(dref.at[wid], idx_v)

        @pl.loop(0, NCH)
        def _(j):
            pltpu.sync_copy(ones_v, acc.at[idx_v.at[j]], add=True)

    plsc.subcore_barrier()
    for r in range(3):
        base = (r * NC + cid) * NPAD + sid * RPT
        pltpu.sync_copy(accs[r].at[pl.ds(sid * RPT, RPT)],
                        out.at[pl.ds(base, RPT)])


@functools.partial(jax.jit, static_argnums=())
def _sc_degree(d0, d1, d2, zn):
    return pl.kernel(
        _sc_degree_body,
        out_type=jax.ShapeDtypeStruct((3 * NC * NPAD,), jnp.float32),
        mesh=_mesh,
        scratch_types=[
            pltpu.VMEM((NCH, C), jnp.int32),
            pltpu.VMEM((C,), jnp.float32),
            pltpu.VMEM_SHARED((NPAD,), jnp.float32),
            pltpu.VMEM_SHARED((NPAD,), jnp.float32),
            pltpu.VMEM_SHARED((NPAD,), jnp.float32),
        ],
    )(d0, d1, d2, zn)


def _sc_edges_body(t0, t1, t2, s0, s1, s2, d0, d1, d2, z2, out,
                   idx_s, idx_d, rows, acc, sem):
    cid = lax.axis_index("c")
    sid = lax.axis_index("s")
    wid = sid * NC + cid
    for r, (tref, sref, dref) in enumerate(((t0, s0, d0), (t1, s1, d1),
                                            (t2, s2, d2))):
        pltpu.sync_copy(z2.at[pl.ds(sid * RPT, RPT)],
                        acc.at[pl.ds(sid * RPT, RPT)])
        pltpu.sync_copy(sref.at[wid], idx_s)
        pltpu.sync_copy(dref.at[wid], idx_d)
        plsc.subcore_barrier()

        @pl.loop(0, NCH)
        def _(j):
            pltpu.async_copy(tref.at[idx_s.at[j]], rows, sem).wait()
            pltpu.sync_copy(rows, acc.at[idx_d.at[j]], add=True)

        plsc.subcore_barrier()
        base = (r * NC + cid) * NPAD + sid * RPT
        pltpu.sync_copy(acc.at[pl.ds(sid * RPT, RPT)],
                        out.at[pl.ds(base, RPT)])


def _sc_edges(t0, t1, t2, s0, s1, s2, d0, d1, d2, z2):
    return pl.kernel(
        _sc_edges_body,
        out_type=jax.ShapeDtypeStruct((3 * NC * NPAD, D), jnp.float32),
        mesh=_mesh,
        scratch_types=[
            pltpu.VMEM((NCH, C), jnp.int32),
            pltpu.VMEM((NCH, C), jnp.int32),
            pltpu.VMEM((C, D), jnp.float32),
            pltpu.VMEM_SHARED((NPAD, D), jnp.float32),
            pltpu.SemaphoreType.DMA,
        ],
    )(t0, t1, t2, s0, s1, s2, d0, d1, d2, z2)


def _dis(degp_ref, r):
    deg = 1.0 + degp_ref[r, 0] + degp_ref[r, 1]
    return lax.rsqrt(deg)


def _tc_prep_body(xp, w0, w1, w2, degp, t0, t1, t2):
    x = xp[...]
    for r, (w, t) in enumerate(((w0, t0), (w1, t1), (w2, t2))):
        dis = _dis(degp, r)
        h = jnp.dot(x, w[...], preferred_element_type=jnp.float32)
        t[...] = h * dis[:, None]


def _tc_prep(xp, w0, w1, w2, degp):
    bs_row = pl.BlockSpec((BN, D), lambda i: (i, 0))
    return pl.pallas_call(
        _tc_prep_body,
        grid=(GRID,),
        in_specs=[
            bs_row,
            pl.BlockSpec((D, D), lambda i: (0, 0)),
            pl.BlockSpec((D, D), lambda i: (0, 0)),
            pl.BlockSpec((D, D), lambda i: (0, 0)),
            pl.BlockSpec((3, NC, BN), lambda i: (0, 0, i)),
        ],
        out_specs=[bs_row, bs_row, bs_row],
        out_shape=[jax.ShapeDtypeStruct((NPAD, D), jnp.float32)] * 3,
    )(xp, w0, w1, w2, degp)


def _tc_mid_body(xp, t0, t1, t2, p, degp, b0, b1, b2, w0, w1, w2,
                 o0, o1, o2):
    x = xp[...]
    ts = (t0, t1, t2)
    bs = (b0, b1, b2)
    diss = [_dis(degp, r) for r in range(3)]
    acc = jnp.zeros((BN, D), jnp.float32)
    for r in range(3):
        agg = (p[r, 0] + p[r, 1] + ts[r][...]) * diss[r][:, None] + bs[r][...]
        acc = acc + agg
    h = x + acc * (1.0 / 3.0)
    h = jnp.where(h >= 0.0, h, 0.01 * h)
    for r, (w, o) in enumerate(((w0, o0), (w1, o1), (w2, o2))):
        hw = jnp.dot(h, w[...], preferred_element_type=jnp.float32)
        o[...] = hw * diss[r][:, None]


def _tc_mid(xp, t0, t1, t2, p, degp, b0, b1, b2, w0, w1, w2):
    bs_row = pl.BlockSpec((BN, D), lambda i: (i, 0))
    bs_w = pl.BlockSpec((D, D), lambda i: (0, 0))
    bs_b = pl.BlockSpec((1, D), lambda i: (0, 0))
    return pl.pallas_call(
        _tc_mid_body,
        grid=(GRID,),
        in_specs=[
            bs_row, bs_row, bs_row, bs_row,
            pl.BlockSpec((3, NC, BN, D), lambda i: (0, 0, i, 0)),
            pl.BlockSpec((3, NC, BN), lambda i: (0, 0, i)),
            bs_b, bs_b, bs_b,
            bs_w, bs_w, bs_w,
        ],
        out_specs=[bs_row, bs_row, bs_row],
        out_shape=[jax.ShapeDtypeStruct((NPAD, D), jnp.float32)] * 3,
    )(xp, t0, t1, t2, p, degp, b0, b1, b2, w0, w1, w2)


def _tc_final_body(t0, t1, t2, p, degp, b0, b1, b2, o):
    ts = (t0, t1, t2)
    bs = (b0, b1, b2)
    acc = jnp.zeros((BN, D), jnp.float32)
    for r in range(3):
        dis = _dis(degp, r)
        agg = (p[r, 0] + p[r, 1] + ts[r][...]) * dis[:, None] + bs[r][...]
        acc = acc + agg
    o[...] = acc * (1.0 / 3.0)


def _tc_final(t0, t1, t2, p, degp, b0, b1, b2):
    bs_row = pl.BlockSpec((BN, D), lambda i: (i, 0))
    bs_b = pl.BlockSpec((1, D), lambda i: (0, 0))
    return pl.pallas_call(
        _tc_final_body,
        grid=(GRID,),
        in_specs=[
            bs_row, bs_row, bs_row,
            pl.BlockSpec((3, NC, BN, D), lambda i: (0, 0, i, 0)),
            pl.BlockSpec((3, NC, BN), lambda i: (0, 0, i)),
            bs_b, bs_b, bs_b,
        ],
        out_specs=bs_row,
        out_shape=jax.ShapeDtypeStruct((NPAD, D), jnp.float32),
    )(t0, t1, t2, p, degp, b0, b1, b2)


def kernel(x, edge_index_parent, edge_index_child, edge_index_relate,
           W1p, b1p, W1c, b1c, W1r, b1r,
           W2p, b2p, W2c, b2c, W2r, b2r):
    xp = jnp.pad(x, ((0, NPAD - N), (0, 0)))
    s0 = edge_index_parent[0].reshape(NW, NCH, C)
    s1 = edge_index_child[0].reshape(NW, NCH, C)
    s2 = edge_index_relate[0].reshape(NW, NCH, C)
    d0 = edge_index_parent[1].reshape(NW, NCH, C)
    d1 = edge_index_child[1].reshape(NW, NCH, C)
    d2 = edge_index_relate[1].reshape(NW, NCH, C)
    zn = jnp.zeros((NPAD,), jnp.float32)
    z2 = jnp.zeros((NPAD, D), jnp.float32)
    b1s = [b.reshape(1, D) for b in (b1p, b1c, b1r)]
    b2s = [b.reshape(1, D) for b in (b2p, b2c, b2r)]

    degp = _sc_degree(d0, d1, d2, zn).reshape(3, NC, NPAD)

    t10, t11, t12 = _tc_prep(xp, W1p, W1c, W1r, degp)
    p1 = _sc_edges(t10, t11, t12, s0, s1, s2, d0, d1, d2, z2)
    p1 = p1.reshape(3, NC, NPAD, D)

    t20, t21, t22 = _tc_mid(xp, t10, t11, t12, p1, degp,
                            b1s[0], b1s[1], b1s[2], W2p, W2c, W2r)
    p2 = _sc_edges(t20, t21, t22, s0, s1, s2, d0, d1, d2, z2)
    p2 = p2.reshape(3, NC, NPAD, D)

    out = _tc_final(t20, t21, t22, p2, degp, b2s[0], b2s[1], b2s[2])
    return out[:N]
